# Initial kernel scaffold; baseline (speedup 1.0000x reference)
import functools
import jax
import jax.numpy as jnp
from jax import lax
from jax.experimental import pallas as pl
from jax.experimental.pallas import tpu as pltpu
from jax.experimental.pallas import tpu_sc as plsc

N = 10000
E = 640000
H = 16
G = 64
ALPHA = 0.2

NP = 10240
NW = 32
CHUNK = 128
EPW = 20480
NCHUNK = EPW // CHUNK
EPAD = NW * EPW
ROWS_PER_SUB = NP // 16


def _pre_body(feat_ref, watt_ref, aatt_ref, w0_ref, x0_ref, m0_ref):
    h = jnp.dot(feat_ref[...], watt_ref[...],
                preferred_element_type=jnp.float32)
    e = jnp.dot(h, aatt_ref[...], preferred_element_type=jnp.float32)
    e = jnp.where(e > 0, e, ALPHA * e)
    att = jax.nn.sigmoid(e)
    v = att * h
    h1 = jnp.where(v > 0, v, jnp.expm1(v))
    x0 = jnp.concatenate(
        [h1, jnp.zeros((NP, G - H), dtype=jnp.float32)], axis=1)
    x0_ref[...] = x0
    m0_ref[...] = jnp.dot(x0, w0_ref[...], preferred_element_type=jnp.float32)


def _pre_call(feat, watt, aatt, w0):
    return pl.pallas_call(
        _pre_body,
        out_shape=[
            jax.ShapeDtypeStruct((NP, G), jnp.float32),
            jax.ShapeDtypeStruct((NP, G), jnp.float32),
        ],
    )(feat, watt, aatt, w0)


_sc_mesh = plsc.VectorSubcoreMesh(core_axis_name="c", subcore_axis_name="s")


@functools.partial(
    pl.kernel,
    out_type=jax.ShapeDtypeStruct((2, NP, G), jnp.float32),
    mesh=_sc_mesh,
    scratch_types=[
        pltpu.VMEM((NCHUNK, CHUNK), jnp.int32),
        pltpu.VMEM((NCHUNK, CHUNK), jnp.int32),
        pltpu.VMEM((CHUNK, G), jnp.float32),
        pltpu.VMEM_SHARED((NP, G), jnp.float32),
        pltpu.SemaphoreType.DMA,
    ],
)
def _seg_sum(m_hbm, src_hbm, dst_hbm, zeros_hbm, out_hbm,
             src_v, dst_v, rows_v, agg_sh, sem):
    c = lax.axis_index("c")
    s = lax.axis_index("s")
    wid = s * 2 + c
    pltpu.sync_copy(zeros_hbm, agg_sh.at[pl.ds(s * ROWS_PER_SUB, ROWS_PER_SUB)])
    pltpu.sync_copy(src_hbm.at[wid], src_v)
    pltpu.sync_copy(dst_hbm.at[wid], dst_v)
    plsc.subcore_barrier()

    def body(j, carry):
        pltpu.async_copy(m_hbm.at[src_v.at[j]], rows_v, sem).wait()
        pltpu.sync_copy(rows_v, agg_sh.at[dst_v.at[j]], add=True)
        return carry

    lax.fori_loop(0, NCHUNK, body, 0)
    plsc.subcore_barrier()
    pltpu.sync_copy(
        agg_sh.at[pl.ds(s * ROWS_PER_SUB, ROWS_PER_SUB)],
        out_hbm.at[c].at[pl.ds(s * ROWS_PER_SUB, ROWS_PER_SUB)])


def _gru_core(part_ref, x_ref, wih_ref, whh_ref, bih_ref, bhh_ref):
    agg = part_ref[0] + part_ref[1]
    gi = jnp.dot(agg, wih_ref[...], preferred_element_type=jnp.float32)
    gi = gi + bih_ref[...]
    gh = jnp.dot(x_ref[...], whh_ref[...], preferred_element_type=jnp.float32)
    gh = gh + bhh_ref[...]
    r = jax.nn.sigmoid(gi[:, :G] + gh[:, :G])
    z = jax.nn.sigmoid(gi[:, G:2 * G] + gh[:, G:2 * G])
    n = jnp.tanh(gi[:, 2 * G:] + r * gh[:, 2 * G:])
    return (1.0 - z) * n + z * x_ref[...]


def _gru_mid_body(part_ref, x_ref, wih_ref, whh_ref, bih_ref, bhh_ref,
                  w1_ref, x1_ref, m1_ref):
    x1 = _gru_core(part_ref, x_ref, wih_ref, whh_ref, bih_ref, bhh_ref)
    x1_ref[...] = x1
    m1_ref[...] = jnp.dot(x1, w1_ref[...], preferred_element_type=jnp.float32)


def _gru_mid_call(part, x, wih_t, whh_t, bih, bhh, w1):
    return pl.pallas_call(
        _gru_mid_body,
        out_shape=[
            jax.ShapeDtypeStruct((NP, G), jnp.float32),
            jax.ShapeDtypeStruct((NP, G), jnp.float32),
        ],
    )(part, x, wih_t, whh_t, bih, bhh, w1)


def _gru_fin_body(part_ref, x_ref, wih_ref, whh_ref, bih_ref, bhh_ref,
                  outw_ref, outb_ref, out_ref):
    x2 = _gru_core(part_ref, x_ref, wih_ref, whh_ref, bih_ref, bhh_ref)
    res = jnp.dot(x2, outw_ref[...], preferred_element_type=jnp.float32)
    res = res + outb_ref[...]
    rows = lax.broadcasted_iota(jnp.int32, (NP, 2), 0)
    res = jnp.where(rows < N, res, 0.0)
    a2 = jnp.sum(res, axis=0, keepdims=True)
    out_ref[...] = jax.nn.softmax(a2, axis=-1)


def _gru_fin_call(part, x, wih_t, whh_t, bih, bhh, outw, outb):
    return pl.pallas_call(
        _gru_fin_body,
        out_shape=jax.ShapeDtypeStruct((1, 2), jnp.float32),
    )(part, x, wih_t, whh_t, bih, bhh, outw, outb)


def kernel(features1, edge_index1, edgesAttr1, adjacency1,
           node2node_features1, W_att, a_att, ggnn_w, gru_wih, gru_whh,
           gru_bih, gru_bhh, out_W, out_b):
    f32 = jnp.float32
    feat = jnp.concatenate(
        [features1.astype(f32),
         jnp.zeros((NP - N, H), dtype=f32)], axis=0)

    src = edge_index1[0].astype(jnp.int32)
    dst = edge_index1[1].astype(jnp.int32)
    pad = EPAD - E
    src_p = jnp.concatenate([src, jnp.zeros((pad,), jnp.int32)])
    dst_p = jnp.concatenate([dst, jnp.full((pad,), NP - 1, jnp.int32)])
    src_p = src_p.reshape(NW, NCHUNK, CHUNK)
    dst_p = dst_p.reshape(NW, NCHUNK, CHUNK)
    zeros_stripe = jnp.zeros((ROWS_PER_SUB, G), f32)

    wih_t = gru_wih.T.astype(f32)
    whh_t = gru_whh.T.astype(f32)
    bih = gru_bih.reshape(1, 3 * G).astype(f32)
    bhh = gru_bhh.reshape(1, 3 * G).astype(f32)
    outb = out_b.reshape(1, 2).astype(f32)

    x0, m0 = _pre_call(feat, W_att.astype(f32), a_att.astype(f32),
                       ggnn_w[0].astype(f32))
    part0 = _seg_sum(m0, src_p, dst_p, zeros_stripe)
    x1, m1 = _gru_mid_call(part0, x0, wih_t, whh_t, bih, bhh,
                           ggnn_w[1].astype(f32))
    part1 = _seg_sum(m1, src_p, dst_p, zeros_stripe)
    out = _gru_fin_call(part1, x1, wih_t, whh_t, bih, bhh,
                        out_W.astype(f32), outb)
    return out


# speedup vs baseline: 5.6000x; 5.6000x over previous
import functools
import jax
import jax.numpy as jnp
from jax import lax
from jax.experimental import pallas as pl
from jax.experimental.pallas import tpu as pltpu
from jax.experimental.pallas import tpu_sc as plsc

N = 10000
E = 640000
H = 16
G = 64
ALPHA = 0.2

NP = 10240
NW = 32
CHUNK = 128
EPW = 20480
NCHUNK = EPW // CHUNK
EPAD = NW * EPW
ROWS_PER_SUB = NP // 16


def _pre_body(feat_ref, watt_ref, aatt_ref, w0_ref, x0_ref, m0_ref):
    h = jnp.dot(feat_ref[...], watt_ref[...],
                preferred_element_type=jnp.float32)
    e = jnp.dot(h, aatt_ref[...], preferred_element_type=jnp.float32)
    e = jnp.where(e > 0, e, ALPHA * e)
    att = jax.nn.sigmoid(e)
    v = att * h
    h1 = jnp.where(v > 0, v, jnp.exp(jnp.minimum(v, 0.0)) - 1.0)
    x0 = jnp.concatenate(
        [h1, jnp.zeros((NP, G - H), dtype=jnp.float32)], axis=1)
    x0_ref[...] = x0
    m0_ref[...] = jnp.dot(x0, w0_ref[...], preferred_element_type=jnp.float32)


def _pre_call(feat, watt, aatt, w0):
    return pl.pallas_call(
        _pre_body,
        out_shape=[
            jax.ShapeDtypeStruct((NP, G), jnp.float32),
            jax.ShapeDtypeStruct((NP, G), jnp.float32),
        ],
    )(feat, watt, aatt, w0)


@functools.lru_cache(maxsize=1)
def _make_seg_sum():
    mesh = plsc.VectorSubcoreMesh(core_axis_name="c", subcore_axis_name="s")

    @functools.partial(
        pl.kernel,
        out_type=jax.ShapeDtypeStruct((2, NP, G), jnp.float32),
        mesh=mesh,
        compiler_params=pltpu.CompilerParams(use_tc_tiling_on_sc=False),
        scratch_types=[
            pltpu.VMEM((NCHUNK, CHUNK), jnp.int32),
            pltpu.VMEM((NCHUNK, CHUNK), jnp.int32),
            pltpu.VMEM((CHUNK, G), jnp.float32),
            pltpu.VMEM_SHARED((NP, G), jnp.float32),
            pltpu.SemaphoreType.DMA,
        ],
    )
    def seg_sum(m_hbm, src_hbm, dst_hbm, zeros_hbm, out_hbm,
                src_v, dst_v, rows_v, agg_sh, sem):
        c = lax.axis_index("c")
        s = lax.axis_index("s")
        wid = s * 2 + c
        pltpu.sync_copy(zeros_hbm,
                        agg_sh.at[pl.ds(s * ROWS_PER_SUB, ROWS_PER_SUB)])
        pltpu.sync_copy(src_hbm.at[wid], src_v)
        pltpu.sync_copy(dst_hbm.at[wid], dst_v)
        plsc.subcore_barrier()

        def body(j, carry):
            pltpu.async_copy(m_hbm.at[src_v.at[j]], rows_v, sem).wait()
            pltpu.sync_copy(rows_v, agg_sh.at[dst_v.at[j]], add=True)
            return carry

        lax.fori_loop(0, NCHUNK, body, 0)
        plsc.subcore_barrier()
        pltpu.sync_copy(
            agg_sh.at[pl.ds(s * ROWS_PER_SUB, ROWS_PER_SUB)],
            out_hbm.at[c].at[pl.ds(s * ROWS_PER_SUB, ROWS_PER_SUB)])

    return seg_sum


def _seg_sum(m, src_p, dst_p, zeros_stripe):
    return _make_seg_sum()(m, src_p, dst_p, zeros_stripe)


def _gru_core(part_ref, x_ref, wih_ref, whh_ref, bih_ref, bhh_ref):
    agg = part_ref[0] + part_ref[1]
    gi = jnp.dot(agg, wih_ref[...], preferred_element_type=jnp.float32)
    gi = gi + bih_ref[...]
    gh = jnp.dot(x_ref[...], whh_ref[...], preferred_element_type=jnp.float32)
    gh = gh + bhh_ref[...]
    r = jax.nn.sigmoid(gi[:, :G] + gh[:, :G])
    z = jax.nn.sigmoid(gi[:, G:2 * G] + gh[:, G:2 * G])
    n = jnp.tanh(gi[:, 2 * G:] + r * gh[:, 2 * G:])
    return (1.0 - z) * n + z * x_ref[...]


def _gru_mid_body(part_ref, x_ref, wih_ref, whh_ref, bih_ref, bhh_ref,
                  w1_ref, x1_ref, m1_ref):
    x1 = _gru_core(part_ref, x_ref, wih_ref, whh_ref, bih_ref, bhh_ref)
    x1_ref[...] = x1
    m1_ref[...] = jnp.dot(x1, w1_ref[...], preferred_element_type=jnp.float32)


def _gru_mid_call(part, x, wih_t, whh_t, bih, bhh, w1):
    return pl.pallas_call(
        _gru_mid_body,
        out_shape=[
            jax.ShapeDtypeStruct((NP, G), jnp.float32),
            jax.ShapeDtypeStruct((NP, G), jnp.float32),
        ],
    )(part, x, wih_t, whh_t, bih, bhh, w1)


def _gru_fin_body(part_ref, x_ref, wih_ref, whh_ref, bih_ref, bhh_ref,
                  outw_ref, outb_ref, out_ref):
    x2 = _gru_core(part_ref, x_ref, wih_ref, whh_ref, bih_ref, bhh_ref)
    res = jnp.dot(x2, outw_ref[...], preferred_element_type=jnp.float32)
    res = res + outb_ref[...]
    rows = lax.broadcasted_iota(jnp.int32, (NP, 2), 0)
    res = jnp.where(rows < N, res, 0.0)
    a2 = jnp.sum(res, axis=0, keepdims=True)
    out_ref[...] = jax.nn.softmax(a2, axis=-1)


def _gru_fin_call(part, x, wih_t, whh_t, bih, bhh, outw, outb):
    return pl.pallas_call(
        _gru_fin_body,
        out_shape=jax.ShapeDtypeStruct((1, 2), jnp.float32),
    )(part, x, wih_t, whh_t, bih, bhh, outw, outb)


def kernel(features1, edge_index1, edgesAttr1, adjacency1,
           node2node_features1, W_att, a_att, ggnn_w, gru_wih, gru_whh,
           gru_bih, gru_bhh, out_W, out_b):
    f32 = jnp.float32
    feat = jnp.concatenate(
        [features1.astype(f32),
         jnp.zeros((NP - N, H), dtype=f32)], axis=0)

    src = edge_index1[0].astype(jnp.int32)
    dst = edge_index1[1].astype(jnp.int32)
    pad = EPAD - E
    src_p = jnp.concatenate([src, jnp.zeros((pad,), jnp.int32)])
    dst_p = jnp.concatenate([dst, jnp.full((pad,), NP - 1, jnp.int32)])
    src_p = src_p.reshape(NW, NCHUNK, CHUNK)
    dst_p = dst_p.reshape(NW, NCHUNK, CHUNK)
    zeros_stripe = jnp.zeros((ROWS_PER_SUB, G), f32)

    wih_t = gru_wih.T.astype(f32)
    whh_t = gru_whh.T.astype(f32)
    bih = gru_bih.reshape(1, 3 * G).astype(f32)
    bhh = gru_bhh.reshape(1, 3 * G).astype(f32)
    outb = out_b.reshape(1, 2).astype(f32)

    x0, m0 = _pre_call(feat, W_att.astype(f32), a_att.astype(f32),
                       ggnn_w[0].astype(f32))
    part0 = _seg_sum(m0, src_p, dst_p, zeros_stripe)
    x1, m1 = _gru_mid_call(part0, x0, wih_t, whh_t, bih, bhh,
                           ggnn_w[1].astype(f32))
    part1 = _seg_sum(m1, src_p, dst_p, zeros_stripe)
    out = _gru_fin_call(part1, x1, wih_t, whh_t, bih, bhh,
                        out_W.astype(f32), outb)
    return out


# speedup vs baseline: 6.6232x; 1.1827x over previous
import functools
import jax
import jax.numpy as jnp
from jax import lax
from jax.experimental import pallas as pl
from jax.experimental.pallas import tpu as pltpu
from jax.experimental.pallas import tpu_sc as plsc

N = 10000
E = 640000
H = 16
G = 64
ALPHA = 0.2

NP = 10240
NW = 32
CHUNK = 128
EPW = 20480
NCHUNK = EPW // CHUNK
EPAD = NW * EPW
ROWS_PER_SUB = NP // 16
NBUF = 4
LOOKAHEAD = 2


def _pre_body(feat_ref, watt_ref, aatt_ref, w0_ref, x0_ref, m0_ref):
    h = jnp.dot(feat_ref[...], watt_ref[...],
                preferred_element_type=jnp.float32)
    e = jnp.dot(h, aatt_ref[...], preferred_element_type=jnp.float32)
    e = jnp.where(e > 0, e, ALPHA * e)
    att = jax.nn.sigmoid(e)
    v = att * h
    h1 = jnp.where(v > 0, v, jnp.exp(jnp.minimum(v, 0.0)) - 1.0)
    x0 = jnp.concatenate(
        [h1, jnp.zeros((NP, G - H), dtype=jnp.float32)], axis=1)
    x0_ref[...] = x0
    m0_ref[...] = jnp.dot(x0, w0_ref[...], preferred_element_type=jnp.float32)


def _pre_call(feat, watt, aatt, w0):
    return pl.pallas_call(
        _pre_body,
        out_shape=[
            jax.ShapeDtypeStruct((NP, G), jnp.float32),
            jax.ShapeDtypeStruct((NP, G), jnp.float32),
        ],
    )(feat, watt, aatt, w0)


@functools.lru_cache(maxsize=1)
def _make_seg_sum():
    mesh = plsc.VectorSubcoreMesh(core_axis_name="c", subcore_axis_name="s")

    @functools.partial(
        pl.kernel,
        out_type=jax.ShapeDtypeStruct((2, NP, G), jnp.float32),
        mesh=mesh,
        compiler_params=pltpu.CompilerParams(use_tc_tiling_on_sc=False),
        scratch_types=[
            pltpu.VMEM((NCHUNK, CHUNK), jnp.int32),
            pltpu.VMEM((NCHUNK, CHUNK), jnp.int32),
            pltpu.VMEM((NBUF, CHUNK, G), jnp.float32),
            pltpu.VMEM_SHARED((NP, G), jnp.float32),
            pltpu.SemaphoreType.DMA((NBUF,)),
            pltpu.SemaphoreType.DMA((NBUF,)),
        ],
    )
    def seg_sum(m_hbm, src_hbm, dst_hbm, zeros_hbm, out_hbm,
                src_v, dst_v, rows_v, agg_sh, gsem, ssem):
        c = lax.axis_index("c")
        s = lax.axis_index("s")
        wid = s * 2 + c
        pltpu.sync_copy(zeros_hbm,
                        agg_sh.at[pl.ds(s * ROWS_PER_SUB, ROWS_PER_SUB)])
        pltpu.sync_copy(src_hbm.at[wid], src_v)
        pltpu.sync_copy(dst_hbm.at[wid], dst_v)
        plsc.subcore_barrier()

        for b in range(LOOKAHEAD):
            pltpu.async_copy(m_hbm.at[src_v.at[b]], rows_v.at[b],
                             gsem.at[b])

        def body(j, carry):
            nj = j + LOOKAHEAD

            @pl.when(nj < NCHUNK)
            def _issue():
                b2 = lax.rem(nj, NBUF)

                @pl.when(nj >= NBUF)
                def _wait_scatter():
                    pltpu.make_async_copy(
                        rows_v.at[b2],
                        agg_sh.at[dst_v.at[nj - NBUF]],
                        ssem.at[b2]).wait()

                pltpu.async_copy(m_hbm.at[src_v.at[nj]], rows_v.at[b2],
                                 gsem.at[b2])

            b = lax.rem(j, NBUF)
            pltpu.make_async_copy(m_hbm.at[src_v.at[j]], rows_v.at[b],
                                  gsem.at[b]).wait()
            pltpu.async_copy(rows_v.at[b], agg_sh.at[dst_v.at[j]],
                             ssem.at[b], add=True)
            return carry

        lax.fori_loop(0, NCHUNK, body, 0)

        for k in range(NBUF):
            j = NCHUNK - NBUF + k
            b = j % NBUF
            pltpu.make_async_copy(rows_v.at[b], agg_sh.at[dst_v.at[j]],
                                  ssem.at[b]).wait()
        plsc.subcore_barrier()
        pltpu.sync_copy(
            agg_sh.at[pl.ds(s * ROWS_PER_SUB, ROWS_PER_SUB)],
            out_hbm.at[c].at[pl.ds(s * ROWS_PER_SUB, ROWS_PER_SUB)])

    return seg_sum


def _seg_sum(m, src_p, dst_p, zeros_stripe):
    return _make_seg_sum()(m, src_p, dst_p, zeros_stripe)


def _gru_core(part_ref, x_ref, wih_ref, whh_ref, bih_ref, bhh_ref):
    agg = part_ref[0] + part_ref[1]
    gi = jnp.dot(agg, wih_ref[...], preferred_element_type=jnp.float32)
    gi = gi + bih_ref[...]
    gh = jnp.dot(x_ref[...], whh_ref[...], preferred_element_type=jnp.float32)
    gh = gh + bhh_ref[...]
    r = jax.nn.sigmoid(gi[:, :G] + gh[:, :G])
    z = jax.nn.sigmoid(gi[:, G:2 * G] + gh[:, G:2 * G])
    n = jnp.tanh(gi[:, 2 * G:] + r * gh[:, 2 * G:])
    return (1.0 - z) * n + z * x_ref[...]


def _gru_mid_body(part_ref, x_ref, wih_ref, whh_ref, bih_ref, bhh_ref,
                  w1_ref, x1_ref, m1_ref):
    x1 = _gru_core(part_ref, x_ref, wih_ref, whh_ref, bih_ref, bhh_ref)
    x1_ref[...] = x1
    m1_ref[...] = jnp.dot(x1, w1_ref[...], preferred_element_type=jnp.float32)


def _gru_mid_call(part, x, wih_t, whh_t, bih, bhh, w1):
    return pl.pallas_call(
        _gru_mid_body,
        out_shape=[
            jax.ShapeDtypeStruct((NP, G), jnp.float32),
            jax.ShapeDtypeStruct((NP, G), jnp.float32),
        ],
    )(part, x, wih_t, whh_t, bih, bhh, w1)


def _gru_fin_body(part_ref, x_ref, wih_ref, whh_ref, bih_ref, bhh_ref,
                  outw_ref, outb_ref, out_ref):
    x2 = _gru_core(part_ref, x_ref, wih_ref, whh_ref, bih_ref, bhh_ref)
    res = jnp.dot(x2, outw_ref[...], preferred_element_type=jnp.float32)
    res = res + outb_ref[...]
    rows = lax.broadcasted_iota(jnp.int32, (NP, 2), 0)
    res = jnp.where(rows < N, res, 0.0)
    a2 = jnp.sum(res, axis=0, keepdims=True)
    out_ref[...] = jax.nn.softmax(a2, axis=-1)


def _gru_fin_call(part, x, wih_t, whh_t, bih, bhh, outw, outb):
    return pl.pallas_call(
        _gru_fin_body,
        out_shape=jax.ShapeDtypeStruct((1, 2), jnp.float32),
    )(part, x, wih_t, whh_t, bih, bhh, outw, outb)


def kernel(features1, edge_index1, edgesAttr1, adjacency1,
           node2node_features1, W_att, a_att, ggnn_w, gru_wih, gru_whh,
           gru_bih, gru_bhh, out_W, out_b):
    f32 = jnp.float32
    feat = jnp.concatenate(
        [features1.astype(f32),
         jnp.zeros((NP - N, H), dtype=f32)], axis=0)

    src = edge_index1[0].astype(jnp.int32)
    dst = edge_index1[1].astype(jnp.int32)
    pad = EPAD - E
    src_p = jnp.concatenate([src, jnp.zeros((pad,), jnp.int32)])
    dst_p = jnp.concatenate([dst, jnp.full((pad,), NP - 1, jnp.int32)])
    src_p = src_p.reshape(NW, NCHUNK, CHUNK)
    dst_p = dst_p.reshape(NW, NCHUNK, CHUNK)
    zeros_stripe = jnp.zeros((ROWS_PER_SUB, G), f32)

    wih_t = gru_wih.T.astype(f32)
    whh_t = gru_whh.T.astype(f32)
    bih = gru_bih.reshape(1, 3 * G).astype(f32)
    bhh = gru_bhh.reshape(1, 3 * G).astype(f32)
    outb = out_b.reshape(1, 2).astype(f32)

    x0, m0 = _pre_call(feat, W_att.astype(f32), a_att.astype(f32),
                       ggnn_w[0].astype(f32))
    part0 = _seg_sum(m0, src_p, dst_p, zeros_stripe)
    x1, m1 = _gru_mid_call(part0, x0, wih_t, whh_t, bih, bhh,
                           ggnn_w[1].astype(f32))
    part1 = _seg_sum(m1, src_p, dst_p, zeros_stripe)
    out = _gru_fin_call(part1, x1, wih_t, whh_t, bih, bhh,
                        out_W.astype(f32), outb)
    return out


# speedup vs baseline: 12.2413x; 1.8482x over previous
import functools
import jax
import jax.numpy as jnp
from jax import lax
from jax.experimental import pallas as pl
from jax.experimental.pallas import tpu as pltpu
from jax.experimental.pallas import tpu_sc as plsc

N = 10000
E = 640000
H = 16
G = 64
ALPHA = 0.2

NP = 10240
NW = 32
CHUNK = 128
EPW = 20480
NCHUNK = EPW // CHUNK
EPAD = NW * EPW
ROWS_PER_SUB = NP // 16
NBUF = 4
LOOKAHEAD = 2


def _pre_body(feat_ref, watt_ref, aatt_ref, h1_ref):
    h = jnp.dot(feat_ref[...], watt_ref[...],
                preferred_element_type=jnp.float32)
    e = jnp.dot(h, aatt_ref[...], preferred_element_type=jnp.float32)
    e = jnp.where(e > 0, e, ALPHA * e)
    att = jax.nn.sigmoid(e)
    v = att * h
    h1_ref[...] = jnp.where(v > 0, v, jnp.exp(jnp.minimum(v, 0.0)) - 1.0)


def _pre_call(feat, watt, aatt):
    return pl.pallas_call(
        _pre_body,
        out_shape=jax.ShapeDtypeStruct((NP, H), jnp.float32),
    )(feat, watt, aatt)


@functools.lru_cache(maxsize=None)
def _make_seg_sum(width):
    mesh = plsc.VectorSubcoreMesh(core_axis_name="c", subcore_axis_name="s")

    @functools.partial(
        pl.kernel,
        out_type=jax.ShapeDtypeStruct((2, NP, width), jnp.float32),
        mesh=mesh,
        compiler_params=pltpu.CompilerParams(use_tc_tiling_on_sc=False),
        scratch_types=[
            pltpu.VMEM((NCHUNK, CHUNK), jnp.int32),
            pltpu.VMEM((NCHUNK, CHUNK), jnp.int32),
            pltpu.VMEM((NBUF, CHUNK, width), jnp.float32),
            pltpu.VMEM_SHARED((NP, width), jnp.float32),
            pltpu.SemaphoreType.DMA((NBUF,)),
            pltpu.SemaphoreType.DMA((NBUF,)),
        ],
    )
    def seg_sum(m_hbm, src_hbm, dst_hbm, zeros_hbm, out_hbm,
                src_v, dst_v, rows_v, agg_sh, gsem, ssem):
        c = lax.axis_index("c")
        s = lax.axis_index("s")
        wid = s * 2 + c
        pltpu.sync_copy(zeros_hbm,
                        agg_sh.at[pl.ds(s * ROWS_PER_SUB, ROWS_PER_SUB)])
        pltpu.sync_copy(src_hbm.at[wid], src_v)
        pltpu.sync_copy(dst_hbm.at[wid], dst_v)
        plsc.subcore_barrier()

        for b in range(LOOKAHEAD):
            pltpu.async_copy(m_hbm.at[src_v.at[b]], rows_v.at[b],
                             gsem.at[b])

        def body(j, carry):
            nj = j + LOOKAHEAD

            @pl.when(nj < NCHUNK)
            def _issue():
                b2 = lax.rem(nj, NBUF)

                @pl.when(nj >= NBUF)
                def _wait_scatter():
                    pltpu.make_async_copy(
                        rows_v.at[b2],
                        agg_sh.at[dst_v.at[nj - NBUF]],
                        ssem.at[b2]).wait()

                pltpu.async_copy(m_hbm.at[src_v.at[nj]], rows_v.at[b2],
                                 gsem.at[b2])

            b = lax.rem(j, NBUF)
            pltpu.make_async_copy(m_hbm.at[src_v.at[j]], rows_v.at[b],
                                  gsem.at[b]).wait()
            pltpu.async_copy(rows_v.at[b], agg_sh.at[dst_v.at[j]],
                             ssem.at[b], add=True)
            return carry

        lax.fori_loop(0, NCHUNK, body, 0)

        for k in range(NBUF):
            j = NCHUNK - NBUF + k
            b = j % NBUF
            pltpu.make_async_copy(rows_v.at[b], agg_sh.at[dst_v.at[j]],
                                  ssem.at[b]).wait()
        plsc.subcore_barrier()
        pltpu.sync_copy(
            agg_sh.at[pl.ds(s * ROWS_PER_SUB, ROWS_PER_SUB)],
            out_hbm.at[c].at[pl.ds(s * ROWS_PER_SUB, ROWS_PER_SUB)])

    return seg_sum


def _seg_sum(m, src_p, dst_p, zeros_stripe):
    return _make_seg_sum(m.shape[-1])(m, src_p, dst_p, zeros_stripe)


def _gru_core(agg, x, wih_ref, whh_ref, bih_ref, bhh_ref):
    gi = jnp.dot(agg, wih_ref[...], preferred_element_type=jnp.float32)
    gi = gi + bih_ref[...]
    gh = jnp.dot(x, whh_ref[...], preferred_element_type=jnp.float32)
    gh = gh + bhh_ref[...]
    r = jax.nn.sigmoid(gi[:, :G] + gh[:, :G])
    z = jax.nn.sigmoid(gi[:, G:2 * G] + gh[:, G:2 * G])
    n = jnp.tanh(gi[:, 2 * G:] + r * gh[:, 2 * G:])
    return (1.0 - z) * n + z * x


def _gru_mid_body(part_ref, h1_ref, w0g_ref, wih_ref, whh_ref, bih_ref,
                  bhh_ref, w1_ref, x1_ref, m1_ref):
    agg16 = part_ref[0] + part_ref[1]
    agg = jnp.dot(agg16, w0g_ref[...], preferred_element_type=jnp.float32)
    x0 = jnp.concatenate(
        [h1_ref[...], jnp.zeros((NP, G - H), dtype=jnp.float32)], axis=1)
    x1 = _gru_core(agg, x0, wih_ref, whh_ref, bih_ref, bhh_ref)
    x1_ref[...] = x1
    m1_ref[...] = jnp.dot(x1, w1_ref[...], preferred_element_type=jnp.float32)


def _gru_mid_call(part, h1, w0g, wih_t, whh_t, bih, bhh, w1):
    return pl.pallas_call(
        _gru_mid_body,
        out_shape=[
            jax.ShapeDtypeStruct((NP, G), jnp.float32),
            jax.ShapeDtypeStruct((NP, G), jnp.float32),
        ],
    )(part, h1, w0g, wih_t, whh_t, bih, bhh, w1)


def _gru_fin_body(part_ref, x_ref, wih_ref, whh_ref, bih_ref, bhh_ref,
                  outw_ref, outb_ref, out_ref):
    x2 = _gru_core(part_ref[0] + part_ref[1], x_ref[...],
                   wih_ref, whh_ref, bih_ref, bhh_ref)
    res = jnp.dot(x2, outw_ref[...], preferred_element_type=jnp.float32)
    res = res + outb_ref[...]
    rows = lax.broadcasted_iota(jnp.int32, (NP, 2), 0)
    res = jnp.where(rows < N, res, 0.0)
    a2 = jnp.sum(res, axis=0, keepdims=True)
    out_ref[...] = jax.nn.softmax(a2, axis=-1)


def _gru_fin_call(part, x, wih_t, whh_t, bih, bhh, outw, outb):
    return pl.pallas_call(
        _gru_fin_body,
        out_shape=jax.ShapeDtypeStruct((1, 2), jnp.float32),
    )(part, x, wih_t, whh_t, bih, bhh, outw, outb)


def kernel(features1, edge_index1, edgesAttr1, adjacency1,
           node2node_features1, W_att, a_att, ggnn_w, gru_wih, gru_whh,
           gru_bih, gru_bhh, out_W, out_b):
    f32 = jnp.float32
    feat = jnp.concatenate(
        [features1.astype(f32),
         jnp.zeros((NP - N, H), dtype=f32)], axis=0)

    src = edge_index1[0].astype(jnp.int32)
    dst = edge_index1[1].astype(jnp.int32)
    pad = EPAD - E
    src_p = jnp.concatenate([src, jnp.zeros((pad,), jnp.int32)])
    dst_p = jnp.concatenate([dst, jnp.full((pad,), NP - 1, jnp.int32)])
    src_p = src_p.reshape(NW, NCHUNK, CHUNK)
    dst_p = dst_p.reshape(NW, NCHUNK, CHUNK)
    zeros_h = jnp.zeros((ROWS_PER_SUB, H), f32)
    zeros_g = jnp.zeros((ROWS_PER_SUB, G), f32)

    wih_t = gru_wih.T.astype(f32)
    whh_t = gru_whh.T.astype(f32)
    bih = gru_bih.reshape(1, 3 * G).astype(f32)
    bhh = gru_bhh.reshape(1, 3 * G).astype(f32)
    outb = out_b.reshape(1, 2).astype(f32)

    h1 = _pre_call(feat, W_att.astype(f32), a_att.astype(f32))
    part0 = _seg_sum(h1, src_p, dst_p, zeros_h)
    w0g = ggnn_w[0][:H, :].astype(f32)
    x1, m1 = _gru_mid_call(part0, h1, w0g, wih_t, whh_t, bih, bhh,
                           ggnn_w[1].astype(f32))
    part1 = _seg_sum(m1, src_p, dst_p, zeros_g)
    out = _gru_fin_call(part1, x1, wih_t, whh_t, bih, bhh,
                        out_W.astype(f32), outb)
    return out


# speedup vs baseline: 27.7160x; 2.2641x over previous
import functools
import jax
import jax.numpy as jnp
from jax import lax
from jax.experimental import pallas as pl
from jax.experimental.pallas import tpu as pltpu
from jax.experimental.pallas import tpu_sc as plsc

N = 10000
E = 640000
H = 16
G = 64
ALPHA = 0.2

NP = 10240
NW = 32
CHUNK = 128
EPW = 20480
NCHUNK = EPW // CHUNK
EPAD = NW * EPW
ROWS_PER_SUB = NP // 16
NBUF = 4
LOOKAHEAD = 2


def _pre_body(feat_ref, watt_ref, aatt_ref, h1_ref):
    h = jnp.dot(feat_ref[...], watt_ref[...],
                preferred_element_type=jnp.float32)
    e = jnp.dot(h, aatt_ref[...], preferred_element_type=jnp.float32)
    e = jnp.where(e > 0, e, ALPHA * e)
    att = jax.nn.sigmoid(e)
    v = att * h
    h1_ref[...] = jnp.where(v > 0, v, jnp.exp(jnp.minimum(v, 0.0)) - 1.0)


def _pre_call(feat, watt, aatt):
    return pl.pallas_call(
        _pre_body,
        out_shape=jax.ShapeDtypeStruct((NP, H), jnp.float32),
    )(feat, watt, aatt)


@functools.lru_cache(maxsize=None)
def _make_seg_sum(width):
    mesh = plsc.VectorSubcoreMesh(core_axis_name="c", subcore_axis_name="s")

    @functools.partial(
        pl.kernel,
        out_type=jax.ShapeDtypeStruct((2, NP, width), jnp.float32),
        mesh=mesh,
        compiler_params=pltpu.CompilerParams(use_tc_tiling_on_sc=False),
        scratch_types=[
            pltpu.VMEM((NCHUNK, CHUNK), jnp.int32),
            pltpu.VMEM((NCHUNK, CHUNK), jnp.int32),
            pltpu.VMEM((NBUF, CHUNK, width), jnp.float32),
            pltpu.VMEM_SHARED((NP, width), jnp.float32),
            pltpu.SemaphoreType.DMA((NBUF,)),
            pltpu.SemaphoreType.DMA((NBUF,)),
        ],
    )
    def seg_sum(m_hbm, src_hbm, dst_hbm, zeros_hbm, out_hbm,
                src_v, dst_v, rows_v, agg_sh, gsem, ssem):
        c = lax.axis_index("c")
        s = lax.axis_index("s")
        wid = s * 2 + c
        pltpu.sync_copy(zeros_hbm,
                        agg_sh.at[pl.ds(s * ROWS_PER_SUB, ROWS_PER_SUB)])
        pltpu.sync_copy(src_hbm.at[wid], src_v)
        pltpu.sync_copy(dst_hbm.at[wid], dst_v)
        plsc.subcore_barrier()

        for b in range(LOOKAHEAD):
            pltpu.async_copy(m_hbm.at[src_v.at[b]], rows_v.at[b],
                             gsem.at[b])

        def body(j, carry):
            nj = j + LOOKAHEAD

            @pl.when(nj < NCHUNK)
            def _issue():
                b2 = lax.rem(nj, NBUF)

                @pl.when(nj >= NBUF)
                def _wait_scatter():
                    pltpu.make_async_copy(
                        rows_v.at[b2],
                        agg_sh.at[dst_v.at[nj - NBUF]],
                        ssem.at[b2]).wait()

                pltpu.async_copy(m_hbm.at[src_v.at[nj]], rows_v.at[b2],
                                 gsem.at[b2])

            b = lax.rem(j, NBUF)
            pltpu.make_async_copy(m_hbm.at[src_v.at[j]], rows_v.at[b],
                                  gsem.at[b]).wait()
            pltpu.async_copy(rows_v.at[b], agg_sh.at[dst_v.at[j]],
                             ssem.at[b], add=True)
            return carry

        lax.fori_loop(0, NCHUNK, body, 0)

        for k in range(NBUF):
            j = NCHUNK - NBUF + k
            b = j % NBUF
            pltpu.make_async_copy(rows_v.at[b], agg_sh.at[dst_v.at[j]],
                                  ssem.at[b]).wait()
        plsc.subcore_barrier()
        pltpu.sync_copy(
            agg_sh.at[pl.ds(s * ROWS_PER_SUB, ROWS_PER_SUB)],
            out_hbm.at[c].at[pl.ds(s * ROWS_PER_SUB, ROWS_PER_SUB)])

    return seg_sum


def _seg_sum(m, src_p, dst_p, zeros_stripe):
    return _make_seg_sum(m.shape[-1])(m, src_p, dst_p, zeros_stripe)


def _gru_core(agg, x, wih_ref, whh_ref, bih_ref, bhh_ref):
    gi = jnp.dot(agg, wih_ref[...], preferred_element_type=jnp.float32)
    gi = gi + bih_ref[...]
    gh = jnp.dot(x, whh_ref[...], preferred_element_type=jnp.float32)
    gh = gh + bhh_ref[...]
    r = jax.nn.sigmoid(gi[:, :G] + gh[:, :G])
    z = jax.nn.sigmoid(gi[:, G:2 * G] + gh[:, G:2 * G])
    n = jnp.tanh(gi[:, 2 * G:] + r * gh[:, 2 * G:])
    return (1.0 - z) * n + z * x


def _gru_mid_body(part_ref, h1_ref, w0g_ref, wih_ref, whh_ref, bih_ref,
                  bhh_ref, w1_ref, x1_ref, m1_ref):
    agg16 = part_ref[0] + part_ref[1]
    agg = jnp.dot(agg16, w0g_ref[...], preferred_element_type=jnp.float32)
    x0 = jnp.concatenate(
        [h1_ref[...], jnp.zeros((NP, G - H), dtype=jnp.float32)], axis=1)
    x1 = _gru_core(agg, x0, wih_ref, whh_ref, bih_ref, bhh_ref)
    x1_ref[...] = x1
    m1_ref[...] = jnp.dot(x1, w1_ref[...], preferred_element_type=jnp.float32)


def _gru_mid_call(part, h1, w0g, wih_t, whh_t, bih, bhh, w1):
    return pl.pallas_call(
        _gru_mid_body,
        out_shape=[
            jax.ShapeDtypeStruct((NP, G), jnp.float32),
            jax.ShapeDtypeStruct((NP, G), jnp.float32),
        ],
    )(part, h1, w0g, wih_t, whh_t, bih, bhh, w1)


def _gru_fin_body(part_ref, x_ref, wih_ref, whh_ref, bih_ref, bhh_ref,
                  outw_ref, outb_ref, out_ref):
    x2 = _gru_core(part_ref[0] + part_ref[1], x_ref[...],
                   wih_ref, whh_ref, bih_ref, bhh_ref)
    res = jnp.dot(x2, outw_ref[...], preferred_element_type=jnp.float32)
    res = res + outb_ref[...]
    rows = lax.broadcasted_iota(jnp.int32, (NP, 2), 0)
    res = jnp.where(rows < N, res, 0.0)
    a2 = jnp.sum(res, axis=0, keepdims=True)
    out_ref[...] = jax.nn.softmax(a2, axis=-1)


def _gru_fin_call(part, x, wih_t, whh_t, bih, bhh, outw, outb):
    return pl.pallas_call(
        _gru_fin_body,
        out_shape=jax.ShapeDtypeStruct((1, 2), jnp.float32),
    )(part, x, wih_t, whh_t, bih, bhh, outw, outb)


def kernel(features1, edge_index1, edgesAttr1, adjacency1,
           node2node_features1, W_att, a_att, ggnn_w, gru_wih, gru_whh,
           gru_bih, gru_bhh, out_W, out_b):
    f32 = jnp.float32
    feat = jnp.concatenate(
        [features1.astype(f32),
         jnp.zeros((NP - N, H), dtype=f32)], axis=0)

    src = edge_index1[0].astype(jnp.int32)
    dst = edge_index1[1].astype(jnp.int32)
    pad = EPAD - E
    pad_ar = jnp.arange(pad, dtype=jnp.int32)
    src_p = jnp.concatenate([src, pad_ar % N])
    dst_p = jnp.concatenate([dst, N + (pad_ar % (NP - N))])
    src_p = src_p.reshape(NW, NCHUNK, CHUNK)
    dst_p = dst_p.reshape(NW, NCHUNK, CHUNK)
    zeros_h = jnp.zeros((ROWS_PER_SUB, H), f32)
    zeros_g = jnp.zeros((ROWS_PER_SUB, G), f32)

    wih_t = gru_wih.T.astype(f32)
    whh_t = gru_whh.T.astype(f32)
    bih = gru_bih.reshape(1, 3 * G).astype(f32)
    bhh = gru_bhh.reshape(1, 3 * G).astype(f32)
    outb = out_b.reshape(1, 2).astype(f32)

    h1 = _pre_call(feat, W_att.astype(f32), a_att.astype(f32))
    part0 = _seg_sum(h1, src_p, dst_p, zeros_h)
    w0g = ggnn_w[0][:H, :].astype(f32)
    x1, m1 = _gru_mid_call(part0, h1, w0g, wih_t, whh_t, bih, bhh,
                           ggnn_w[1].astype(f32))
    part1 = _seg_sum(m1, src_p, dst_p, zeros_g)
    out = _gru_fin_call(part1, x1, wih_t, whh_t, bih, bhh,
                        out_W.astype(f32), outb)
    return out


# speedup vs baseline: 29.4839x; 1.0638x over previous
import functools
import jax
import jax.numpy as jnp
from jax import lax
from jax.experimental import pallas as pl
from jax.experimental.pallas import tpu as pltpu
from jax.experimental.pallas import tpu_sc as plsc

N = 10000
E = 640000
H = 16
G = 64
ALPHA = 0.2

NP = 10240
NW = 32
CHUNK = 128
EPW = 20480
NCHUNK = EPW // CHUNK
EPAD = NW * EPW
ROWS_PER_SUB = NP // 16
_RING = {16: (8, 5), 64: (4, 2)}


def _pre_body(feat_ref, watt_ref, aatt_ref, h1_ref):
    h = jnp.dot(feat_ref[...], watt_ref[...],
                preferred_element_type=jnp.float32)
    e = jnp.dot(h, aatt_ref[...], preferred_element_type=jnp.float32)
    e = jnp.where(e > 0, e, ALPHA * e)
    att = jax.nn.sigmoid(e)
    v = att * h
    h1_ref[:N, :] = jnp.where(v > 0, v, jnp.exp(jnp.minimum(v, 0.0)) - 1.0)
    h1_ref[N:, :] = jnp.zeros((NP - N, H), jnp.float32)


def _pre_call(feat, watt, aatt):
    return pl.pallas_call(
        _pre_body,
        out_shape=jax.ShapeDtypeStruct((NP, H), jnp.float32),
    )(feat, watt, aatt)


def _gh_body(x_ref, whh_ref, bhh_ref, gh_ref):
    gh_ref[...] = jnp.dot(x_ref[...], whh_ref[...],
                          preferred_element_type=jnp.float32) + bhh_ref[...]


def _gh_call(x, whh_t, bhh):
    return pl.pallas_call(
        _gh_body,
        out_shape=jax.ShapeDtypeStruct((NP, 3 * G), jnp.float32),
    )(x, whh_t, bhh)


@functools.lru_cache(maxsize=None)
def _make_seg_sum(width):
    NBUF, LOOKAHEAD = _RING[width]
    mesh = plsc.VectorSubcoreMesh(core_axis_name="c", subcore_axis_name="s")

    @functools.partial(
        pl.kernel,
        out_type=jax.ShapeDtypeStruct((2, NP, width), jnp.float32),
        mesh=mesh,
        compiler_params=pltpu.CompilerParams(use_tc_tiling_on_sc=False),
        scratch_types=[
            pltpu.VMEM((NCHUNK, CHUNK), jnp.int32),
            pltpu.VMEM((NCHUNK, CHUNK), jnp.int32),
            pltpu.VMEM((NBUF, CHUNK, width), jnp.float32),
            pltpu.VMEM_SHARED((NP, width), jnp.float32),
            pltpu.SemaphoreType.DMA((NBUF,)),
            pltpu.SemaphoreType.DMA((NBUF,)),
        ],
    )
    def seg_sum(m_hbm, src_hbm, dst_hbm, zeros_hbm, out_hbm,
                src_v, dst_v, rows_v, agg_sh, gsem, ssem):
        c = lax.axis_index("c")
        s = lax.axis_index("s")
        wid = s * 2 + c
        pltpu.sync_copy(zeros_hbm,
                        agg_sh.at[pl.ds(s * ROWS_PER_SUB, ROWS_PER_SUB)])
        pltpu.sync_copy(src_hbm.at[wid], src_v)
        pltpu.sync_copy(dst_hbm.at[wid], dst_v)
        plsc.subcore_barrier()

        for b in range(LOOKAHEAD):
            pltpu.async_copy(m_hbm.at[src_v.at[b]], rows_v.at[b],
                             gsem.at[b])

        def body(j, carry):
            nj = j + LOOKAHEAD

            @pl.when(nj < NCHUNK)
            def _issue():
                b2 = lax.rem(nj, NBUF)

                @pl.when(nj >= NBUF)
                def _wait_scatter():
                    pltpu.make_async_copy(
                        rows_v.at[b2],
                        agg_sh.at[dst_v.at[nj - NBUF]],
                        ssem.at[b2]).wait()

                pltpu.async_copy(m_hbm.at[src_v.at[nj]], rows_v.at[b2],
                                 gsem.at[b2])

            b = lax.rem(j, NBUF)
            pltpu.make_async_copy(m_hbm.at[src_v.at[j]], rows_v.at[b],
                                  gsem.at[b]).wait()
            pltpu.async_copy(rows_v.at[b], agg_sh.at[dst_v.at[j]],
                             ssem.at[b], add=True)
            return carry

        lax.fori_loop(0, NCHUNK, body, 0)

        for k in range(NBUF):
            j = NCHUNK - NBUF + k
            b = j % NBUF
            pltpu.make_async_copy(rows_v.at[b], agg_sh.at[dst_v.at[j]],
                                  ssem.at[b]).wait()
        plsc.subcore_barrier()
        pltpu.sync_copy(
            agg_sh.at[pl.ds(s * ROWS_PER_SUB, ROWS_PER_SUB)],
            out_hbm.at[c].at[pl.ds(s * ROWS_PER_SUB, ROWS_PER_SUB)])

    return seg_sum


def _seg_sum(m, src_p, dst_p, zeros_stripe):
    return _make_seg_sum(m.shape[-1])(m, src_p, dst_p, zeros_stripe)


def _gru_gates(gi, gh, x):
    r = jax.nn.sigmoid(gi[:, :G] + gh[:, :G])
    z = jax.nn.sigmoid(gi[:, G:2 * G] + gh[:, G:2 * G])
    n = jnp.tanh(gi[:, 2 * G:] + r * gh[:, 2 * G:])
    return (1.0 - z) * n + z * x


def _gru_mid_body(part_ref, h1_ref, gh_ref, w0g_ref, wih_ref, bih_ref,
                  w1_ref, x1_ref, m1_ref):
    agg16 = part_ref[0] + part_ref[1]
    w16 = jnp.dot(w0g_ref[...], wih_ref[...],
                  preferred_element_type=jnp.float32)
    gi = jnp.dot(agg16, w16, preferred_element_type=jnp.float32)
    gi = gi + bih_ref[...]
    x0 = jnp.concatenate(
        [h1_ref[...], jnp.zeros((NP, G - H), dtype=jnp.float32)], axis=1)
    x1 = _gru_gates(gi, gh_ref[...], x0)
    x1_ref[...] = x1
    m1_ref[...] = jnp.dot(x1, w1_ref[...], preferred_element_type=jnp.float32)


def _gru_mid_call(part, h1, gh, w0g, wih_t, bih, w1):
    return pl.pallas_call(
        _gru_mid_body,
        out_shape=[
            jax.ShapeDtypeStruct((NP, G), jnp.float32),
            jax.ShapeDtypeStruct((NP, G), jnp.float32),
        ],
    )(part, h1, gh, w0g, wih_t, bih, w1)


def _gru_fin_body(part_ref, x_ref, gh_ref, wih_ref, bih_ref,
                  outw_ref, outb_ref, out_ref):
    agg = part_ref[0] + part_ref[1]
    gi = jnp.dot(agg, wih_ref[...], preferred_element_type=jnp.float32)
    gi = gi + bih_ref[...]
    x2 = _gru_gates(gi, gh_ref[...], x_ref[...])
    res = jnp.dot(x2, outw_ref[...], preferred_element_type=jnp.float32)
    res = res + outb_ref[...]
    rows = lax.broadcasted_iota(jnp.int32, (NP, 2), 0)
    res = jnp.where(rows < N, res, 0.0)
    a2 = jnp.sum(res, axis=0, keepdims=True)
    out_ref[...] = jax.nn.softmax(a2, axis=-1)


def _gru_fin_call(part, x, gh, wih_t, bih, outw, outb):
    return pl.pallas_call(
        _gru_fin_body,
        out_shape=jax.ShapeDtypeStruct((1, 2), jnp.float32),
    )(part, x, gh, wih_t, bih, outw, outb)


def kernel(features1, edge_index1, edgesAttr1, adjacency1,
           node2node_features1, W_att, a_att, ggnn_w, gru_wih, gru_whh,
           gru_bih, gru_bhh, out_W, out_b):
    f32 = jnp.float32
    src = edge_index1[0].astype(jnp.int32)
    dst = edge_index1[1].astype(jnp.int32)
    pad = EPAD - E
    pad_ar = jnp.arange(pad, dtype=jnp.int32)
    src_p = jnp.concatenate([src, pad_ar % N])
    dst_p = jnp.concatenate([dst, N + (pad_ar % (NP - N))])
    src_p = src_p.reshape(NW, NCHUNK, CHUNK)
    dst_p = dst_p.reshape(NW, NCHUNK, CHUNK)
    zeros_h = jnp.zeros((ROWS_PER_SUB, H), f32)
    zeros_g = jnp.zeros((ROWS_PER_SUB, G), f32)

    wih_t = gru_wih.T.astype(f32)
    whh_t = gru_whh.T.astype(f32)
    bih = gru_bih.reshape(1, 3 * G).astype(f32)
    bhh = gru_bhh.reshape(1, 3 * G).astype(f32)
    outb = out_b.reshape(1, 2).astype(f32)

    h1 = _pre_call(features1.astype(f32), W_att.astype(f32),
                   a_att.astype(f32))
    part0 = _seg_sum(h1, src_p, dst_p, zeros_h)
    gh0 = _gh_call(h1, whh_t[:H, :], bhh)
    w0g = ggnn_w[0][:H, :].astype(f32)
    x1, m1 = _gru_mid_call(part0, h1, gh0, w0g, wih_t, bih,
                           ggnn_w[1].astype(f32))
    part1 = _seg_sum(m1, src_p, dst_p, zeros_g)
    gh1 = _gh_call(x1, whh_t, bhh)
    out = _gru_fin_call(part1, x1, gh1, wih_t, bih,
                        out_W.astype(f32), outb)
    return out


# speedup vs baseline: 34.5343x; 1.1713x over previous
import functools
import jax
import jax.numpy as jnp
from jax import lax
from jax.experimental import pallas as pl
from jax.experimental.pallas import tpu as pltpu
from jax.experimental.pallas import tpu_sc as plsc

N = 10000
E = 640000
H = 16
G = 64
ALPHA = 0.2

NP = 10240
NW = 32
EPW = E // NW
TAIL = 32
ROWS_PER_SUB = NP // 16
_CFG = {16: (512, 8, 4), 64: (192, 3, 2)}


def _pre_body(feat_ref, watt_ref, aatt_ref, h1_ref):
    h = jnp.dot(feat_ref[...], watt_ref[...],
                preferred_element_type=jnp.float32)
    e = jnp.dot(h, aatt_ref[...], preferred_element_type=jnp.float32)
    e = jnp.where(e > 0, e, ALPHA * e)
    att = jax.nn.sigmoid(e)
    v = att * h
    h1_ref[:N, :] = jnp.where(v > 0, v, jnp.exp(jnp.minimum(v, 0.0)) - 1.0)
    h1_ref[N:, :] = jnp.zeros((NP - N, H), jnp.float32)


def _pre_call(feat, watt, aatt):
    return pl.pallas_call(
        _pre_body,
        out_shape=jax.ShapeDtypeStruct((NP, H), jnp.float32),
    )(feat, watt, aatt)


def _gh_body(x_ref, whh_ref, bhh_ref, gh_ref):
    gh_ref[...] = jnp.dot(x_ref[...], whh_ref[...],
                          preferred_element_type=jnp.float32) + bhh_ref[...]


def _gh_call(x, whh_t, bhh):
    return pl.pallas_call(
        _gh_body,
        out_shape=jax.ShapeDtypeStruct((NP, 3 * G), jnp.float32),
    )(x, whh_t, bhh)


@functools.lru_cache(maxsize=None)
def _make_seg_sum(width):
    CHUNK, NBUF, LOOKAHEAD = _CFG[width]
    NCHUNK = (EPW - TAIL) // CHUNK
    mesh = plsc.VectorSubcoreMesh(core_axis_name="c", subcore_axis_name="s")

    @functools.partial(
        pl.kernel,
        out_type=jax.ShapeDtypeStruct((2, NP, 128), jnp.float32),
        mesh=mesh,
        compiler_params=pltpu.CompilerParams(use_tc_tiling_on_sc=False),
        scratch_types=[
            pltpu.VMEM((2, EPW), jnp.int32),
            pltpu.VMEM((NBUF, CHUNK, width), jnp.float32),
            pltpu.VMEM_SHARED((NP, width), jnp.float32),
            pltpu.SemaphoreType.DMA((NBUF,)),
            pltpu.SemaphoreType.DMA((NBUF,)),
        ],
    )
    def seg_sum(m_hbm, eix_hbm, zeros_hbm, out_hbm,
                idx_v, rows_v, agg_sh, gsem, ssem):
        c = lax.axis_index("c")
        s = lax.axis_index("s")
        wid = s * 2 + c
        base = wid * EPW
        pltpu.sync_copy(zeros_hbm,
                        agg_sh.at[pl.ds(s * ROWS_PER_SUB, ROWS_PER_SUB)])
        pltpu.sync_copy(eix_hbm.at[:, pl.ds(base, EPW)], idx_v)
        plsc.subcore_barrier()

        for b in range(LOOKAHEAD):
            pltpu.async_copy(
                m_hbm.at[idx_v.at[0, pl.ds(b * CHUNK, CHUNK)]],
                rows_v.at[b], gsem.at[b])

        def body(j, carry):
            nj = j + LOOKAHEAD

            @pl.when(nj < NCHUNK)
            def _issue():
                b2 = lax.rem(nj, NBUF)

                @pl.when(nj >= NBUF)
                def _wait_scatter():
                    pltpu.make_async_copy(
                        rows_v.at[b2],
                        agg_sh.at[idx_v.at[1, pl.ds((nj - NBUF) * CHUNK,
                                                    CHUNK)]],
                        ssem.at[b2]).wait()

                pltpu.async_copy(
                    m_hbm.at[idx_v.at[0, pl.ds(nj * CHUNK, CHUNK)]],
                    rows_v.at[b2], gsem.at[b2])

            b = lax.rem(j, NBUF)
            pltpu.make_async_copy(
                m_hbm.at[idx_v.at[0, pl.ds(j * CHUNK, CHUNK)]],
                rows_v.at[b], gsem.at[b]).wait()
            pltpu.async_copy(
                rows_v.at[b],
                agg_sh.at[idx_v.at[1, pl.ds(j * CHUNK, CHUNK)]],
                ssem.at[b], add=True)
            return carry

        lax.fori_loop(0, NCHUNK, body, 0)

        for k in range(NBUF):
            j = NCHUNK - NBUF + k
            b = j % NBUF
            pltpu.make_async_copy(
                rows_v.at[b],
                agg_sh.at[idx_v.at[1, pl.ds(j * CHUNK, CHUNK)]],
                ssem.at[b]).wait()

        toff = NCHUNK * CHUNK
        pltpu.async_copy(
            m_hbm.at[idx_v.at[0, pl.ds(toff, TAIL)]],
            rows_v.at[0, pl.ds(0, TAIL)], gsem.at[0]).wait()
        pltpu.sync_copy(
            rows_v.at[0, pl.ds(0, TAIL)],
            agg_sh.at[idx_v.at[1, pl.ds(toff, TAIL)]], add=True)

        plsc.subcore_barrier()
        pltpu.sync_copy(
            agg_sh.at[pl.ds(s * ROWS_PER_SUB, ROWS_PER_SUB)],
            out_hbm.at[c, pl.ds(s * ROWS_PER_SUB, ROWS_PER_SUB),
                       pl.ds(0, width)])

    return seg_sum


def _seg_sum(m, eix, zeros_stripe):
    return _make_seg_sum(m.shape[-1])(m, eix, zeros_stripe)


def _gru_gates(gi, gh, x):
    r = jax.nn.sigmoid(gi[:, :G] + gh[:, :G])
    z = jax.nn.sigmoid(gi[:, G:2 * G] + gh[:, G:2 * G])
    n = jnp.tanh(gi[:, 2 * G:] + r * gh[:, 2 * G:])
    return (1.0 - z) * n + z * x


RB = 2048
NRB = NP // RB


def _gru_mid_body(part_ref, h1_ref, gh_ref, w0g_ref, wih_ref, bih_ref,
                  w1_ref, x1_ref, m1_ref):
    agg16 = part_ref[0, :, :H] + part_ref[1, :, :H]
    w16 = jnp.dot(w0g_ref[...], wih_ref[...],
                  preferred_element_type=jnp.float32)
    gi = jnp.dot(agg16, w16, preferred_element_type=jnp.float32)
    gi = gi + bih_ref[...]
    x0 = jnp.concatenate(
        [h1_ref[...], jnp.zeros((RB, G - H), dtype=jnp.float32)], axis=1)
    x1 = _gru_gates(gi, gh_ref[...], x0)
    x1_ref[...] = x1
    m1_ref[...] = jnp.dot(x1, w1_ref[...], preferred_element_type=jnp.float32)


def _gru_mid_call(part, h1, gh, w0g, wih_t, bih, w1):
    return pl.pallas_call(
        _gru_mid_body,
        grid=(NRB,),
        in_specs=[
            pl.BlockSpec((2, RB, 128), lambda i: (0, i, 0)),
            pl.BlockSpec((RB, H), lambda i: (i, 0)),
            pl.BlockSpec((RB, 3 * G), lambda i: (i, 0)),
            pl.BlockSpec((H, G), lambda i: (0, 0)),
            pl.BlockSpec((G, 3 * G), lambda i: (0, 0)),
            pl.BlockSpec((1, 3 * G), lambda i: (0, 0)),
            pl.BlockSpec((G, G), lambda i: (0, 0)),
        ],
        out_specs=[
            pl.BlockSpec((RB, G), lambda i: (i, 0)),
            pl.BlockSpec((RB, G), lambda i: (i, 0)),
        ],
        out_shape=[
            jax.ShapeDtypeStruct((NP, G), jnp.float32),
            jax.ShapeDtypeStruct((NP, G), jnp.float32),
        ],
    )(part, h1, gh, w0g, wih_t, bih, w1)


def _gru_fin_body(part_ref, x_ref, gh_ref, wih_ref, bih_ref,
                  outw_ref, outb_ref, out_ref):
    i = pl.program_id(0)
    agg = part_ref[0, :, :G] + part_ref[1, :, :G]
    gi = jnp.dot(agg, wih_ref[...], preferred_element_type=jnp.float32)
    gi = gi + bih_ref[...]
    x2 = _gru_gates(gi, gh_ref[...], x_ref[...])
    res = jnp.dot(x2, outw_ref[...], preferred_element_type=jnp.float32)
    res = res + outb_ref[...]
    rows = lax.broadcasted_iota(jnp.int32, (RB, 2), 0) + i * RB
    res = jnp.where(rows < N, res, 0.0)
    a2 = jnp.sum(res, axis=0, keepdims=True)

    @pl.when(i == 0)
    def _init():
        out_ref[...] = a2

    @pl.when(i > 0)
    def _acc():
        out_ref[...] = out_ref[...] + a2

    @pl.when(i == NRB - 1)
    def _fin():
        out_ref[...] = jax.nn.softmax(out_ref[...], axis=-1)


def _gru_fin_call(part, x, gh, wih_t, bih, outw, outb):
    return pl.pallas_call(
        _gru_fin_body,
        grid=(NRB,),
        in_specs=[
            pl.BlockSpec((2, RB, 128), lambda i: (0, i, 0)),
            pl.BlockSpec((RB, G), lambda i: (i, 0)),
            pl.BlockSpec((RB, 3 * G), lambda i: (i, 0)),
            pl.BlockSpec((G, 3 * G), lambda i: (0, 0)),
            pl.BlockSpec((1, 3 * G), lambda i: (0, 0)),
            pl.BlockSpec((G, 2), lambda i: (0, 0)),
            pl.BlockSpec((1, 2), lambda i: (0, 0)),
        ],
        out_specs=pl.BlockSpec((1, 2), lambda i: (0, 0)),
        out_shape=jax.ShapeDtypeStruct((1, 2), jnp.float32),
    )(part, x, gh, wih_t, bih, outw, outb)


def kernel(features1, edge_index1, edgesAttr1, adjacency1,
           node2node_features1, W_att, a_att, ggnn_w, gru_wih, gru_whh,
           gru_bih, gru_bhh, out_W, out_b):
    f32 = jnp.float32
    eix = edge_index1.astype(jnp.int32)
    zeros_h = jnp.zeros((ROWS_PER_SUB, H), f32)
    zeros_g = jnp.zeros((ROWS_PER_SUB, G), f32)

    wih_t = gru_wih.T.astype(f32)
    whh_t = gru_whh.T.astype(f32)
    bih = gru_bih.reshape(1, 3 * G).astype(f32)
    bhh = gru_bhh.reshape(1, 3 * G).astype(f32)
    outb = out_b.reshape(1, 2).astype(f32)

    h1 = _pre_call(features1.astype(f32), W_att.astype(f32),
                   a_att.astype(f32))
    part0 = _seg_sum(h1, eix, zeros_h)
    gh0 = _gh_call(h1, whh_t[:H, :], bhh)
    w0g = ggnn_w[0][:H, :].astype(f32)
    x1, m1 = _gru_mid_call(part0, h1, gh0, w0g, wih_t, bih,
                           ggnn_w[1].astype(f32))
    part1 = _seg_sum(m1, eix, zeros_g)
    gh1 = _gh_call(x1, whh_t, bhh)
    out = _gru_fin_call(part1, x1, gh1, wih_t, bih,
                        out_W.astype(f32), outb)
    return out
